# baseline (speedup 1.0000x reference)
import jax
import jax.numpy as jnp
from jax import lax
from jax.experimental import pallas as pl
from jax.experimental.pallas import tpu as pltpu
from jax.experimental.pallas import tpu_sc as plsc

NC = 2
NS = 16
NW = NC * NS
L = 16

N = 100000
K = 16
IN_DIM = 28
EMB_DIM = 128
W = K + 1

C = 112
NCH = 28
B_PER_W = C * NCH
LAST_BASE = N - C
R = 4


def _body(x_hbm, table_hbm, out_hbm, table_s,
          x_v0, x_v1, x_v2, x_v3, idx_v0, idx_v1, idx_v2, idx_v3,
          out_v0, out_v1, out_v2, out_v3,
          sx0, sx1, sx2, sx3, sg0, sg1, sg2, sg3, sw0, sw1, sw2, sw3):
    s = lax.axis_index("s")
    c = lax.axis_index("c")
    base = (s * NC + c) * B_PER_W
    lane = lax.iota(jnp.int32, L)
    xv = [x_v0, x_v1, x_v2, x_v3]
    iv = [idx_v0, idx_v1, idx_v2, idx_v3]
    ov = [out_v0, out_v1, out_v2, out_v3]
    sx = [sx0, sx1, sx2, sx3]
    sg = [sg0, sg1, sg2, sg3]
    sw = [sw0, sw1, sw2, sw3]

    def row0(i):
        return jnp.minimum(base + i * C, LAST_BASE)

    def x_copy(i, b):
        return pltpu.make_async_copy(x_hbm.at[pl.ds(row0(i), C)], xv[b], sx[b])

    def g_copy(b):
        return pltpu.make_async_copy(table_hbm.at[iv[b]], ov[b], sg[b])

    def w_copy(i, b):
        return pltpu.make_async_copy(ov[b], out_hbm.at[pl.ds(row0(i), C)], sw[b])

    def extract(b):
        def idx_grp(g, _):
            rows = lane + g * L
            vals = plsc.load_gather(xv[b], [rows, jnp.full((L,), K, jnp.int32)])
            iv[b][pl.ds(g * L, L)] = vals.astype(jnp.int32)
            return 0

        lax.fori_loop(0, C // L, idx_grp, 0, unroll=True)

    def struct(b):
        def struct_row(r, _):
            ov[b][r, pl.ds(0, K)] = xv[b][r, pl.ds(0, K)]
            return 0

        lax.fori_loop(0, C, struct_row, 0, unroll=8)

    for i in range(R):
        x_copy(i, i).start()

    @pl.when(s == 0)
    def _():
        pltpu.sync_copy(table_hbm, table_s)

    plsc.subcore_barrier()
    x_copy(0, 0).wait()
    extract(0)
    g_copy(0).start()
    x_copy(1, 1).wait()
    extract(1)
    g_copy(1).start()

    def body(i, j, with_wb_wait):
        b2 = (j + 2) % R
        b = j % R
        x_copy(i + 2, b2).wait()
        extract(b2)
        if with_wb_wait:
            w_copy(i - 2, b2).wait()
        g_copy(b2).start()
        g_copy(b).wait()
        struct(b)
        w_copy(i, b).start()
        x_copy(i + R, b).start()

    for i in range(2):
        body(i, i, with_wb_wait=False)

    def quad(q, _):
        for j in range(R):
            body(q * R + 2 + j, (2 + j) % R, with_wb_wait=True)
        return 0

    lax.fori_loop(0, (NCH - R) // R, quad, 0)

    for i in range(NCH - 2, NCH):
        b = i % R
        g_copy(b).wait()
        struct(b)
        w_copy(i, b).start()
    for i in range(NCH - R, NCH):
        w_copy(i, i % R).wait()
    for i in range(NCH, NCH + 2):
        x_copy(i, i % R).wait()


@jax.jit
def _run(x, table):
    table128 = jnp.pad(table, ((0, 0), (K, 0)))
    mesh = plsc.VectorSubcoreMesh(
        core_axis_name="c", subcore_axis_name="s", num_cores=NC, num_subcores=NS
    )
    return pl.kernel(
        _body,
        out_type=jax.ShapeDtypeStruct((N, EMB_DIM), jnp.float32),
        mesh=mesh,
        compiler_params=pltpu.CompilerParams(needs_layout_passes=False),
        scratch_types=[
            pltpu.VMEM_SHARED((IN_DIM, EMB_DIM), jnp.float32),
        ]
        + [pltpu.VMEM((C, W), jnp.float32)] * 4
        + [pltpu.VMEM((C,), jnp.int32)] * 4
        + [pltpu.VMEM((C, EMB_DIM), jnp.float32)] * 4
        + [pltpu.SemaphoreType.DMA] * 12,
    )(x, table128)


def kernel(x, table):
    return _run(x, table)


# speedup vs baseline: 2.1341x; 2.1341x over previous
import jax
import jax.numpy as jnp
from jax import lax
from jax.experimental import pallas as pl
from jax.experimental.pallas import tpu as pltpu
from jax.experimental.pallas import tpu_sc as plsc

NC = 2
NS = 16
NW = NC * NS
L = 16

N = 100000
K = 16
IN_DIM = 28
EMB_DIM = 128
W = K + 1

C = 112
G = 48
NCH = 28
B_PER_W = C * NCH
LAST_BASE = N - C
R = 4


def _body(x_hbm, table_hbm, out_hbm, table_s, table_v,
          x_v0, x_v1, x_v2, x_v3, idx_v0, idx_v1, idx_v2, idx_v3,
          out_v0, out_v1, out_v2, out_v3,
          sx0, sx1, sx2, sx3, sg0, sg1, sg2, sg3, sw0, sw1, sw2, sw3):
    s = lax.axis_index("s")
    c = lax.axis_index("c")
    base = (s * NC + c) * B_PER_W
    lane = lax.iota(jnp.int32, L)
    xv = [x_v0, x_v1, x_v2, x_v3]
    iv = [idx_v0, idx_v1, idx_v2, idx_v3]
    ov = [out_v0, out_v1, out_v2, out_v3]
    sx = [sx0, sx1, sx2, sx3]
    sg = [sg0, sg1, sg2, sg3]
    sw = [sw0, sw1, sw2, sw3]

    def row0(i):
        return jnp.minimum(base + i * C, LAST_BASE)

    def x_copy(i, b):
        return pltpu.make_async_copy(x_hbm.at[pl.ds(row0(i), C)], xv[b], sx[b])

    def g_copy(b):
        return pltpu.make_async_copy(table_s.at[iv[b]], ov[b].at[pl.ds(0, G)],
                                     sg[b])

    def w_copy(i, b):
        return pltpu.make_async_copy(ov[b], out_hbm.at[pl.ds(row0(i), C)], sw[b])

    def extract(b):
        def idx_grp(g, _):
            rows = lane + g * L
            vals = plsc.load_gather(xv[b], [rows, jnp.full((L,), K, jnp.int32)])
            iv[b][pl.ds(g * L, L)] = vals.astype(jnp.int32)
            return 0

        lax.fori_loop(0, G // L, idx_grp, 0, unroll=True)

    def struct(b):
        def struct_row(r, _):
            ov[b][r, pl.ds(0, K)] = xv[b][r, pl.ds(0, K)]
            return 0

        lax.fori_loop(0, C, struct_row, 0, unroll=8)

    def emb(b):
        def emb_row(r, _):
            t = xv[b][r, pl.ds(1, L)][L - 1].astype(jnp.int32)
            for cc in range(0, EMB_DIM - K, L):
                ov[b][r, pl.ds(K + cc, L)] = table_v[t, pl.ds(K + cc, L)]
            return 0

        lax.fori_loop(G, C, emb_row, 0, unroll=4)

    for i in range(R):
        x_copy(i, i).start()

    pltpu.sync_copy(table_hbm, table_v)

    @pl.when(s == 0)
    def _():
        pltpu.sync_copy(table_hbm, table_s)

    plsc.subcore_barrier()
    x_copy(0, 0).wait()
    extract(0)
    g_copy(0).start()
    x_copy(1, 1).wait()
    extract(1)
    g_copy(1).start()

    def body(i, j, with_wb_wait):
        b2 = (j + 2) % R
        b = j % R
        x_copy(i + 2, b2).wait()
        extract(b2)
        if with_wb_wait:
            w_copy(i - 2, b2).wait()
        g_copy(b2).start()
        emb(b)
        g_copy(b).wait()
        struct(b)
        w_copy(i, b).start()
        x_copy(i + R, b).start()

    for i in range(2):
        body(i, i, with_wb_wait=False)

    def quad(q, _):
        for j in range(R):
            body(q * R + 2 + j, (2 + j) % R, with_wb_wait=True)
        return 0

    lax.fori_loop(0, (NCH - R) // R, quad, 0)

    for i in range(NCH - 2, NCH):
        b = i % R
        emb(b)
        g_copy(b).wait()
        struct(b)
        w_copy(i, b).start()
    for i in range(NCH - R, NCH):
        w_copy(i, i % R).wait()
    for i in range(NCH, NCH + 2):
        x_copy(i, i % R).wait()


@jax.jit
def _run(x, table):
    table128 = jnp.pad(table, ((0, 0), (K, 0)))
    mesh = plsc.VectorSubcoreMesh(
        core_axis_name="c", subcore_axis_name="s", num_cores=NC, num_subcores=NS
    )
    return pl.kernel(
        _body,
        out_type=jax.ShapeDtypeStruct((N, EMB_DIM), jnp.float32),
        mesh=mesh,
        compiler_params=pltpu.CompilerParams(needs_layout_passes=False),
        scratch_types=[
            pltpu.VMEM_SHARED((IN_DIM, EMB_DIM), jnp.float32),
            pltpu.VMEM((IN_DIM, EMB_DIM), jnp.float32),
        ]
        + [pltpu.VMEM((C, W), jnp.float32)] * 4
        + [pltpu.VMEM((G,), jnp.int32)] * 4
        + [pltpu.VMEM((C, EMB_DIM), jnp.float32)] * 4
        + [pltpu.SemaphoreType.DMA] * 12,
    )(x, table128)


def kernel(x, table):
    return _run(x, table)


# speedup vs baseline: 2.3262x; 1.0900x over previous
import jax
import jax.numpy as jnp
from jax import lax
from jax.experimental import pallas as pl
from jax.experimental.pallas import tpu as pltpu
from jax.experimental.pallas import tpu_sc as plsc

NC = 2
NS = 16
NW = NC * NS
L = 16

N = 100000
K = 16
IN_DIM = 28
EMB_DIM = 128
W = K + 1

C = 112
G = 48
NCH = 28
B_PER_W = C * NCH
LAST_BASE = N - C
R = 4


def _body(x_hbm, table_hbm, out_hbm, table_s, table_v,
          x_v0, x_v1, x_v2, x_v3, idx_v0, idx_v1, idx_v2, idx_v3,
          out_v0, out_v1, out_v2, out_v3,
          sx0, sx1, sx2, sx3, sg0, sg1, sg2, sg3, sw0, sw1, sw2, sw3):
    s = lax.axis_index("s")
    c = lax.axis_index("c")
    base = (s * NC + c) * B_PER_W
    lane = lax.iota(jnp.int32, L)
    xv = [x_v0, x_v1, x_v2, x_v3]
    iv = [idx_v0, idx_v1, idx_v2, idx_v3]
    ov = [out_v0, out_v1, out_v2, out_v3]
    sx = [sx0, sx1, sx2, sx3]
    sg = [sg0, sg1, sg2, sg3]
    sw = [sw0, sw1, sw2, sw3]

    def row0(i):
        return jnp.minimum(base + i * C, LAST_BASE)

    def x_copy(i, b):
        return pltpu.make_async_copy(x_hbm.at[pl.ds(row0(i), C)], xv[b], sx[b])

    def g_copy(b):
        return pltpu.make_async_copy(table_s.at[iv[b].at[pl.ds(0, G)]],
                                     ov[b].at[pl.ds(0, G)], sg[b])

    def w_copy(i, b):
        return pltpu.make_async_copy(ov[b], out_hbm.at[pl.ds(row0(i), C)], sw[b])

    def extract(b):
        def idx_grp(g, _):
            rows = lane + g * L
            vals = plsc.load_gather(xv[b], [rows, jnp.full((L,), K, jnp.int32)])
            iv[b][pl.ds(g * L, L)] = vals.astype(jnp.int32)
            return 0

        lax.fori_loop(0, C // L, idx_grp, 0, unroll=True)

    def struct(b):
        def struct_row(r, _):
            ov[b][r, pl.ds(0, K)] = xv[b][r, pl.ds(0, K)]
            return 0

        lax.fori_loop(0, C, struct_row, 0, unroll=8)

    def emb(b):
        def emb_row(r, _):
            idxv = plsc.load_gather(iv[b], [jnp.full((L,), r, jnp.int32)])
            for cc in range(0, EMB_DIM - K, L):
                vals = plsc.load_gather(table_v, [idxv, lane + K + cc])
                ov[b][r, pl.ds(K + cc, L)] = vals
            return 0

        lax.fori_loop(G, C, emb_row, 0, unroll=4)

    for i in range(R):
        x_copy(i, i).start()

    pltpu.sync_copy(table_hbm, table_v)

    @pl.when(s == 0)
    def _():
        pltpu.sync_copy(table_hbm, table_s)

    plsc.subcore_barrier()
    x_copy(0, 0).wait()
    extract(0)
    g_copy(0).start()
    x_copy(1, 1).wait()
    extract(1)
    g_copy(1).start()

    def body(i, j, with_wb_wait):
        b2 = (j + 2) % R
        b = j % R
        x_copy(i + 2, b2).wait()
        extract(b2)
        if with_wb_wait:
            w_copy(i - 2, b2).wait()
        g_copy(b2).start()
        emb(b)
        g_copy(b).wait()
        struct(b)
        w_copy(i, b).start()
        x_copy(i + R, b).start()

    for i in range(2):
        body(i, i, with_wb_wait=False)

    def quad(q, _):
        for j in range(R):
            body(q * R + 2 + j, (2 + j) % R, with_wb_wait=True)
        return 0

    lax.fori_loop(0, (NCH - R) // R, quad, 0)

    for i in range(NCH - 2, NCH):
        b = i % R
        emb(b)
        g_copy(b).wait()
        struct(b)
        w_copy(i, b).start()
    for i in range(NCH - R, NCH):
        w_copy(i, i % R).wait()
    for i in range(NCH, NCH + 2):
        x_copy(i, i % R).wait()


@jax.jit
def _run(x, table):
    table128 = jnp.pad(table, ((0, 0), (K, 0)))
    mesh = plsc.VectorSubcoreMesh(
        core_axis_name="c", subcore_axis_name="s", num_cores=NC, num_subcores=NS
    )
    return pl.kernel(
        _body,
        out_type=jax.ShapeDtypeStruct((N, EMB_DIM), jnp.float32),
        mesh=mesh,
        compiler_params=pltpu.CompilerParams(needs_layout_passes=False),
        scratch_types=[
            pltpu.VMEM_SHARED((IN_DIM, EMB_DIM), jnp.float32),
            pltpu.VMEM((IN_DIM, EMB_DIM), jnp.float32),
        ]
        + [pltpu.VMEM((C, W), jnp.float32)] * 4
        + [pltpu.VMEM((C,), jnp.int32)] * 4
        + [pltpu.VMEM((C, EMB_DIM), jnp.float32)] * 4
        + [pltpu.SemaphoreType.DMA] * 12,
    )(x, table128)


def kernel(x, table):
    return _run(x, table)


# speedup vs baseline: 3.3974x; 1.4605x over previous
import jax
import jax.numpy as jnp
from jax import lax
from jax.experimental import pallas as pl
from jax.experimental.pallas import tpu as pltpu
from jax.experimental.pallas import tpu_sc as plsc

NC = 2
NS = 16
NW = NC * NS
L = 16

N = 100000
K = 16
IN_DIM = 28
EMB_DIM = 128
W = K + 1

C = 112
NCH = 28
B_PER_W = C * NCH
LAST_BASE = N - C
R = 4


def _body(x_hbm, table_hbm, out_hbm, table_s, t_in, t_out,
          x_v0, x_v1, x_v2, x_v3, idx_v0, idx_v1, idx_v2, idx_v3,
          out_v0, out_v1, out_v2, out_v3,
          sx0, sx1, sx2, sx3, sg0, sg1, sg2, sg3, sw0, sw1, sw2, sw3):
    s = lax.axis_index("s")
    c = lax.axis_index("c")
    base = (s * NC + c) * B_PER_W
    lane = lax.iota(jnp.int32, L)
    xv = [x_v0, x_v1, x_v2, x_v3]
    iv = [idx_v0, idx_v1, idx_v2, idx_v3]
    ov = [out_v0, out_v1, out_v2, out_v3]
    sx = [sx0, sx1, sx2, sx3]
    sg = [sg0, sg1, sg2, sg3]
    sw = [sw0, sw1, sw2, sw3]

    def row0(i):
        return jnp.minimum(base + i * C, LAST_BASE)

    def x_copy(i, b):
        return pltpu.make_async_copy(x_hbm.at[pl.ds(row0(i), C)], xv[b], sx[b])

    def g_copy(b):
        return pltpu.make_async_copy(table_s.at[iv[b]], ov[b], sg[b])

    def w_copy(i, b):
        return pltpu.make_async_copy(ov[b], out_hbm.at[pl.ds(row0(i), C)], sw[b])

    def extract(b):
        def idx_grp(g, _):
            rows = lane + g * L
            vals = plsc.load_gather(xv[b], [rows, jnp.full((L,), K, jnp.int32)])
            iv[b][pl.ds(g * L, L)] = vals.astype(jnp.int32)
            return 0

        lax.fori_loop(0, C // L, idx_grp, 0, unroll=True)

    def struct(b):
        def struct_row(r, _):
            ov[b][r, pl.ds(0, K)] = xv[b][r, pl.ds(0, K)]
            return 0

        lax.fori_loop(0, C, struct_row, 0, unroll=8)

    for i in range(R):
        x_copy(i, i).start()

    @pl.when(s == 0)
    def _():
        pltpu.sync_copy(table_hbm, t_in)
        zeros = jnp.zeros((L,), jnp.float32)

        def pad_row(r, _):
            t_out[r, pl.ds(0, K)] = zeros
            for cc in range(0, EMB_DIM - K, L):
                t_out[r, pl.ds(K + cc, L)] = t_in[r, pl.ds(cc, L)]
            return 0

        lax.fori_loop(0, IN_DIM, pad_row, 0)
        pltpu.sync_copy(t_out, table_s)

    plsc.subcore_barrier()
    x_copy(0, 0).wait()
    extract(0)
    g_copy(0).start()
    x_copy(1, 1).wait()
    extract(1)
    g_copy(1).start()

    def body(i, j, with_wb_wait):
        b2 = (j + 2) % R
        b = j % R
        x_copy(i + 2, b2).wait()
        extract(b2)
        if with_wb_wait:
            w_copy(i - 2, b2).wait()
        g_copy(b2).start()
        g_copy(b).wait()
        struct(b)
        w_copy(i, b).start()
        x_copy(i + R, b).start()

    for i in range(2):
        body(i, i, with_wb_wait=False)

    def quad(q, _):
        for j in range(R):
            body(q * R + 2 + j, (2 + j) % R, with_wb_wait=True)
        return 0

    lax.fori_loop(0, (NCH - R) // R, quad, 0)

    for i in range(NCH - 2, NCH):
        b = i % R
        g_copy(b).wait()
        struct(b)
        w_copy(i, b).start()
    for i in range(NCH - R, NCH):
        w_copy(i, i % R).wait()
    for i in range(NCH, NCH + 2):
        x_copy(i, i % R).wait()


@jax.jit
def _run(x, table):
    mesh = plsc.VectorSubcoreMesh(
        core_axis_name="c", subcore_axis_name="s", num_cores=NC, num_subcores=NS
    )
    return pl.kernel(
        _body,
        out_type=jax.ShapeDtypeStruct((N, EMB_DIM), jnp.float32),
        mesh=mesh,
        compiler_params=pltpu.CompilerParams(needs_layout_passes=False),
        scratch_types=[
            pltpu.VMEM_SHARED((IN_DIM, EMB_DIM), jnp.float32),
            pltpu.VMEM((IN_DIM, EMB_DIM - K), jnp.float32),
            pltpu.VMEM((IN_DIM, EMB_DIM), jnp.float32),
        ]
        + [pltpu.VMEM((C, W), jnp.float32)] * 4
        + [pltpu.VMEM((C,), jnp.int32)] * 4
        + [pltpu.VMEM((C, EMB_DIM), jnp.float32)] * 4
        + [pltpu.SemaphoreType.DMA] * 12,
    )(x, table)


def kernel(x, table):
    return _run(x, table)
